# baseline (speedup 1.0000x reference)
import functools

import jax
import jax.numpy as jnp
from jax import lax
from jax.experimental import pallas as pl
from jax.experimental.pallas import tpu as pltpu
from jax.experimental.pallas import tpu_sc as plsc

_NC = 2
_NS = 16
_NW = _NC * _NS


def _gather_stage(emb, first_w_flat, idx_flat):
    n, = idx_flat.shape
    v, d = emb.shape
    bpw = n // _NW
    mesh = plsc.VectorSubcoreMesh(core_axis_name="c", subcore_axis_name="s")

    @functools.partial(
        pl.kernel,
        mesh=mesh,
        compiler_params=pltpu.CompilerParams(use_tc_tiling_on_sc=False),
        out_type=(jax.ShapeDtypeStruct((n, d), jnp.float32),
                  jax.ShapeDtypeStruct((n,), jnp.float32)),
        scratch_types=[
            pltpu.VMEM((bpw,), jnp.int32),
            pltpu.VMEM((bpw, d), jnp.float32),
            pltpu.VMEM((bpw,), jnp.float32),
            pltpu.SemaphoreType.DMA,
            pltpu.SemaphoreType.DMA,
        ],
    )
    def sc_kernel(emb_hbm, fw_hbm, idx_hbm, out_emb_hbm, out_fw_hbm,
                  idx_v, rows_v, fw_v, sem_e, sem_f):
        wid = lax.axis_index("s") * _NC + lax.axis_index("c")
        base = wid * bpw
        pltpu.sync_copy(idx_hbm.at[pl.ds(base, bpw)], idx_v)
        ce = pltpu.async_copy(emb_hbm.at[idx_v], rows_v, sem_e)
        cf = pltpu.async_copy(fw_hbm.at[idx_v], fw_v, sem_f)
        ce.wait()
        cf.wait()
        pltpu.sync_copy(rows_v, out_emb_hbm.at[pl.ds(base, bpw)])
        pltpu.sync_copy(fw_v, out_fw_hbm.at[pl.ds(base, bpw)])

    return sc_kernel(emb, first_w_flat, idx_flat)


def _bn_relu(h):
    m = jnp.mean(h, axis=0, keepdims=True)
    c = h - m
    var = jnp.mean(c * c, axis=0, keepdims=True)
    return jnp.maximum(c / jnp.sqrt(var + 1e-5), 0.0)


def _dense_body(x_ref, yf_ref, s_ref, w1_ref, b1_ref, w2_ref, b2_ref,
                w3_ref, b3_ref, fc1_ref, fc2_ref, fc3_ref, fcb_ref, out_ref):
    hi = lax.Precision.HIGHEST
    x = x_ref[...]
    s_mat = s_ref[...]
    summed = jnp.dot(x, s_mat, precision=hi)
    sq_sum = jnp.dot(x * x, s_mat, precision=hi)
    y_secd = 0.5 * (summed * summed - sq_sum)

    h = jnp.dot(x, w1_ref[...], precision=hi) + b1_ref[...]
    h = _bn_relu(h)
    h = jnp.dot(h, w2_ref[...], precision=hi) + b2_ref[...]
    h = _bn_relu(h)
    h = jnp.dot(h, w3_ref[...], precision=hi) + b3_ref[...]
    h = _bn_relu(h)

    out = (jnp.sum(yf_ref[...] * fc1_ref[...], axis=1, keepdims=True)
           + jnp.sum(y_secd * fc2_ref[...], axis=1, keepdims=True)
           + jnp.sum(h * fc3_ref[...], axis=1, keepdims=True)
           + fcb_ref[...])
    out_ref[...] = out


def kernel(feat_index, first_w, emb, W1, b1, W2, b2, W3, b3, fcW, fcb):
    b, f = feat_index.shape
    v, d = emb.shape
    h_dim = b1.shape[0]

    idx_flat = feat_index.astype(jnp.int32).reshape(-1)
    fw_flat = first_w[jnp.arange(v, dtype=jnp.int32), 0]
    rows, fw = _gather_stage(emb, fw_flat, idx_flat)
    x = rows.reshape(b, f * d)
    yf = fw.reshape(b, f)

    s_mat = (jnp.arange(f * d, dtype=jnp.int32)[:, None] % d
             == jnp.arange(d, dtype=jnp.int32)[None, :]).astype(jnp.float32)

    out = pl.pallas_call(
        _dense_body,
        out_shape=jax.ShapeDtypeStruct((b, 1), jnp.float32),
    )(x, yf, s_mat, W1, b1.reshape(1, h_dim), W2, b2.reshape(1, h_dim),
      W3, b3.reshape(1, h_dim),
      fcW[:f, 0].reshape(1, f), fcW[f:f + d, 0].reshape(1, d),
      fcW[f + d:, 0].reshape(1, h_dim), fcb.reshape(1, 1))
    return out.reshape(b)


# speedup vs baseline: 1.0007x; 1.0007x over previous
import functools

import jax
import jax.numpy as jnp
from jax import lax
from jax.experimental import pallas as pl
from jax.experimental.pallas import tpu as pltpu
from jax.experimental.pallas import tpu_sc as plsc

_NC = 2
_NS = 16
_NW = _NC * _NS


def _gather_stage(emb, first_w_flat, idx_flat, idx_fw):
    n, = idx_flat.shape
    v, d = emb.shape
    bpw = n // _NW
    mesh = plsc.VectorSubcoreMesh(core_axis_name="c", subcore_axis_name="s")

    @functools.partial(
        pl.kernel,
        mesh=mesh,
        compiler_params=pltpu.CompilerParams(use_tc_tiling_on_sc=False),
        out_type=(jax.ShapeDtypeStruct((n, d), jnp.float32),
                  jax.ShapeDtypeStruct((n,), jnp.float32)),
        scratch_types=[
            pltpu.VMEM((bpw,), jnp.int32),
            pltpu.VMEM((bpw,), jnp.int32),
            pltpu.VMEM((bpw, d), jnp.float32),
            pltpu.VMEM((bpw,), jnp.float32),
            pltpu.SemaphoreType.DMA,
            pltpu.SemaphoreType.DMA,
        ],
    )
    def sc_kernel(emb_hbm, fw_hbm, idx_hbm, idxf_hbm, out_emb_hbm, out_fw_hbm,
                  idx_v, idxf_v, rows_v, fw_v, sem_e, sem_f):
        wid = lax.axis_index("s") * _NC + lax.axis_index("c")
        base = wid * bpw
        pltpu.sync_copy(idx_hbm.at[pl.ds(base, bpw)], idx_v)
        pltpu.sync_copy(idxf_hbm.at[pl.ds(base, bpw)], idxf_v)
        ce = pltpu.async_copy(emb_hbm.at[idx_v], rows_v, sem_e)
        cf = pltpu.async_copy(fw_hbm.at[idxf_v], fw_v, sem_f)
        ce.wait()
        cf.wait()
        pltpu.sync_copy(rows_v, out_emb_hbm.at[pl.ds(base, bpw)])
        pltpu.sync_copy(fw_v, out_fw_hbm.at[pl.ds(base, bpw)])

    return sc_kernel(emb, first_w_flat, idx_flat, idx_fw)


def _bn_relu(h):
    m = jnp.mean(h, axis=0, keepdims=True)
    c = h - m
    var = jnp.mean(c * c, axis=0, keepdims=True)
    return jnp.maximum(c / jnp.sqrt(var + 1e-5), 0.0)


def _dense_body(x_ref, yf_ref, s_ref, w1_ref, b1_ref, w2_ref, b2_ref,
                w3_ref, b3_ref, fc1_ref, fc2_ref, fc3_ref, fcb_ref, out_ref):
    hi = lax.Precision.HIGHEST
    x = x_ref[...]
    s_mat = s_ref[...]
    summed = jnp.dot(x, s_mat, precision=hi)
    sq_sum = jnp.dot(x * x, s_mat, precision=hi)
    y_secd = 0.5 * (summed * summed - sq_sum)

    h = jnp.dot(x, w1_ref[...], precision=hi) + b1_ref[...]
    h = _bn_relu(h)
    h = jnp.dot(h, w2_ref[...], precision=hi) + b2_ref[...]
    h = _bn_relu(h)
    h = jnp.dot(h, w3_ref[...], precision=hi) + b3_ref[...]
    h = _bn_relu(h)

    out = (jnp.sum(yf_ref[...] * fc1_ref[...], axis=1, keepdims=True)
           + jnp.sum(y_secd * fc2_ref[...], axis=1, keepdims=True)
           + jnp.sum(h * fc3_ref[...], axis=1, keepdims=True)
           + fcb_ref[...])
    out_ref[...] = out


def kernel(feat_index, first_w, emb, W1, b1, W2, b2, W3, b3, fcW, fcb):
    b, f = feat_index.shape
    v, d = emb.shape
    h_dim = b1.shape[0]

    idx_flat = feat_index.astype(jnp.int32).reshape(-1)
    fw_perm = first_w[jnp.arange(v, dtype=jnp.int32) ^ 1, 0]
    rows, fw = _gather_stage(emb, fw_perm, idx_flat, idx_flat ^ 1)
    x = rows.reshape(b, f * d)
    yf = fw.reshape(b, f)

    s_mat = (jnp.arange(f * d, dtype=jnp.int32)[:, None] % d
             == jnp.arange(d, dtype=jnp.int32)[None, :]).astype(jnp.float32)

    out = pl.pallas_call(
        _dense_body,
        out_shape=jax.ShapeDtypeStruct((b, 1), jnp.float32),
    )(x, yf, s_mat, W1, b1.reshape(1, h_dim), W2, b2.reshape(1, h_dim),
      W3, b3.reshape(1, h_dim),
      fcW[:f, 0].reshape(1, f), fcW[f:f + d, 0].reshape(1, d),
      fcW[f + d:, 0].reshape(1, h_dim), fcb.reshape(1, 1))
    return out.reshape(b)


# speedup vs baseline: 1.1249x; 1.1242x over previous
import functools

import jax
import jax.numpy as jnp
from jax import lax
from jax.experimental import pallas as pl
from jax.experimental.pallas import tpu as pltpu
from jax.experimental.pallas import tpu_sc as plsc

_NC = 2
_NS = 16
_NW = _NC * _NS


def _gather_stage(emb, first_w_flat, idx_flat):
    n, = idx_flat.shape
    v, d = emb.shape
    bpw = n // _NW
    mesh = plsc.VectorSubcoreMesh(core_axis_name="c", subcore_axis_name="s")

    @functools.partial(
        pl.kernel,
        mesh=mesh,
        compiler_params=pltpu.CompilerParams(use_tc_tiling_on_sc=False),
        out_type=(jax.ShapeDtypeStruct((n, d), jnp.float32),
                  jax.ShapeDtypeStruct((n,), jnp.float32)),
        scratch_types=[
            pltpu.VMEM((bpw,), jnp.int32),
            pltpu.VMEM((bpw, d), jnp.float32),
            pltpu.VMEM((bpw,), jnp.float32),
            pltpu.SemaphoreType.DMA,
            pltpu.SemaphoreType.DMA,
        ],
    )
    def sc_kernel(emb_hbm, fw_hbm, idx_hbm, out_emb_hbm, out_fw_hbm,
                  idx_v, rows_v, fw_v, sem_e, sem_f):
        wid = lax.axis_index("s") * _NC + lax.axis_index("c")
        base = wid * bpw
        pltpu.sync_copy(idx_hbm.at[pl.ds(base, bpw)], idx_v)
        ce = pltpu.async_copy(emb_hbm.at[idx_v], rows_v, sem_e)
        cf = pltpu.async_copy(fw_hbm.at[idx_v], fw_v, sem_f)
        ce.wait()
        cf.wait()
        pltpu.sync_copy(rows_v, out_emb_hbm.at[pl.ds(base, bpw)])
        pltpu.sync_copy(fw_v, out_fw_hbm.at[pl.ds(base, bpw)])

    return sc_kernel(emb, first_w_flat, idx_flat)


def _bn_relu(h):
    m = jnp.mean(h, axis=0, keepdims=True)
    c = h - m
    var = jnp.mean(c * c, axis=0, keepdims=True)
    return jnp.maximum(c / jnp.sqrt(var + 1e-5), 0.0)


def _dense_body(x_ref, yf_ref, s_ref, w1_ref, b1_ref, w2_ref, b2_ref,
                w3_ref, b3_ref, fc1_ref, fc2_ref, fc3_ref, fcb_ref, out_ref):
    hi = lax.Precision.DEFAULT
    x = x_ref[...]
    s_mat = s_ref[...]
    summed = jnp.dot(x, s_mat, precision=hi)
    sq_sum = jnp.dot(x * x, s_mat, precision=hi)
    y_secd = 0.5 * (summed * summed - sq_sum)

    h = jnp.dot(x, w1_ref[...], precision=hi) + b1_ref[...]
    h = _bn_relu(h)
    h = jnp.dot(h, w2_ref[...], precision=hi) + b2_ref[...]
    h = _bn_relu(h)
    h = jnp.dot(h, w3_ref[...], precision=hi) + b3_ref[...]
    h = _bn_relu(h)

    out = (jnp.sum(yf_ref[...] * fc1_ref[...], axis=1, keepdims=True)
           + jnp.sum(y_secd * fc2_ref[...], axis=1, keepdims=True)
           + jnp.sum(h * fc3_ref[...], axis=1, keepdims=True)
           + fcb_ref[...])
    out_ref[...] = out


def kernel(feat_index, first_w, emb, W1, b1, W2, b2, W3, b3, fcW, fcb):
    b, f = feat_index.shape
    v, d = emb.shape
    h_dim = b1.shape[0]

    idx_flat = feat_index.astype(jnp.int32).reshape(-1)
    rows, fw = _gather_stage(emb, first_w.reshape(-1), idx_flat)
    x = rows.reshape(b, f * d)
    yf = fw.reshape(b, f)

    s_mat = (jnp.arange(f * d, dtype=jnp.int32)[:, None] % d
             == jnp.arange(d, dtype=jnp.int32)[None, :]).astype(jnp.float32)

    out = pl.pallas_call(
        _dense_body,
        out_shape=jax.ShapeDtypeStruct((b, 1), jnp.float32),
    )(x, yf, s_mat, W1, b1.reshape(1, h_dim), W2, b2.reshape(1, h_dim),
      W3, b3.reshape(1, h_dim),
      fcW[:f, 0].reshape(1, f), fcW[f:f + d, 0].reshape(1, d),
      fcW[f + d:, 0].reshape(1, h_dim), fcb.reshape(1, 1))
    return out.reshape(b)
